# baseline (speedup 1.0000x reference)
import functools

import jax
import jax.numpy as jnp
from jax import lax
from jax.experimental import pallas as pl
from jax.experimental.pallas import tpu as pltpu
from jax.experimental.pallas import tpu_sc as plsc

N = 10000
D = 128
NC = 2
NS = 16
NW = NC * NS
CHUNK = 128
SEG_CPT = 16
C0_FRAC = 0.9
SLAB = 632
LAST = N - (NS - 1) * SLAB


SEG = 32


@functools.lru_cache(maxsize=None)
def _make_seg_kernel(cpt):
    mesh = plsc.VectorSubcoreMesh(core_axis_name="c", subcore_axis_name="s")
    nseg = cpt // SEG
    npair = (nseg - 1) // 2
    idx_t = pltpu.VMEM((SEG, CHUNK), jnp.int32)
    row_t = pltpu.VMEM((CHUNK, D), jnp.float32)

    @functools.partial(
        pl.kernel,
        mesh=mesh,
        out_type=jax.ShapeDtypeStruct((N, D), jnp.float32),
        scratch_types=(
            [pltpu.VMEM_SHARED((N + 8, D), jnp.float32)]
            + [idx_t] * 4
            + [row_t] * 2
            + [pltpu.SemaphoreType.DMA] * 4
        ),
    )
    def seg(h_hbm, src_hbm, dst_hbm, out_hbm, acc,
            sia, dia, sib, dib, ra, rb, ia, ib, ga, gb):
        c = lax.axis_index("c")
        s = lax.axis_index("s")
        slab_off = pl.multiple_of(s * SLAB, 8)

        @pl.when((c == 0) & (s < NS - 1))
        def _():
            pltpu.sync_copy(h_hbm.at[pl.ds(slab_off, SLAB)],
                            acc.at[pl.ds(slab_off, SLAB)])

        @pl.when((c == 0) & (s == NS - 1))
        def _():
            pltpu.sync_copy(h_hbm.at[pl.ds((NS - 1) * SLAB, LAST)],
                            acc.at[pl.ds((NS - 1) * SLAB, LAST)])

        plsc.subcore_barrier()

        @pl.when(c == 0)
        def _():
            base = s * cpt

            def fetch_idx(g, si, di, sem):
                row = pl.multiple_of(base + g * SEG, 8)
                pltpu.async_copy(src_hbm.at[pl.ds(row, SEG)], si, sem)
                pltpu.async_copy(dst_hbm.at[pl.ds(row, SEG)], di, sem)

            def wait_idx(si, di, sem):
                pltpu.make_async_copy(src_hbm.at[pl.ds(0, SEG)], si, sem).wait()
                pltpu.make_async_copy(dst_hbm.at[pl.ds(0, SEG)], di, sem).wait()

            def run_inner(si, di):
                pltpu.async_copy(h_hbm.at[si.at[0]], ra, ga)

                @pl.loop(0, SEG, step=2)
                def _(j):
                    pltpu.async_copy(h_hbm.at[si.at[j + 1]], rb, gb)
                    pltpu.make_async_copy(h_hbm.at[si.at[j]], ra, ga).wait()
                    pltpu.sync_copy(ra, acc.at[di.at[j]], add=True)

                    @pl.when(j + 2 < SEG)
                    def _():
                        pltpu.async_copy(h_hbm.at[si.at[j + 2]], ra, ga)

                    pltpu.make_async_copy(h_hbm.at[si.at[j + 1]], rb, gb).wait()
                    pltpu.sync_copy(rb, acc.at[di.at[j + 1]], add=True)

            fetch_idx(0, sia, dia, ia)
            fetch_idx(1, sib, dib, ib)

            @pl.loop(0, npair)
            def _(p):
                wait_idx(sia, dia, ia)
                run_inner(sia, dia)
                fetch_idx(2 * p + 2, sia, dia, ia)
                wait_idx(sib, dib, ib)
                run_inner(sib, dib)
                fetch_idx(2 * p + 3, sib, dib, ib)

            wait_idx(sia, dia, ia)
            run_inner(sia, dia)
            wait_idx(sib, dib, ib)

        plsc.subcore_barrier()

        @pl.when((c == 0) & (s < NS - 1))
        def _():
            pltpu.sync_copy(acc.at[pl.ds(slab_off, SLAB)],
                            out_hbm.at[pl.ds(slab_off, SLAB)])

        @pl.when((c == 0) & (s == NS - 1))
        def _():
            pltpu.sync_copy(acc.at[pl.ds((NS - 1) * SLAB, LAST)],
                            out_hbm.at[pl.ds((NS - 1) * SLAB, LAST)])

    return seg


def _segment_partials(h, srcp, dstp, cpt):
    return _make_seg_kernel(cpt)(h, srcp, dstp)


def _mm(a, w):
    return lax.dot_general(a, w, (((1,), (1,)), ((), ())),
                           preferred_element_type=jnp.float32,
                           precision=lax.Precision.HIGHEST)


def _mlp_bn(p_ref, x, wa_ref, ba_ref, wb_ref, bb_ref, g_ref, b_ref):
    y = p_ref[...]
    t = jnp.maximum(_mm(y, wa_ref[...]) + ba_ref[...], 0.0)
    z = _mm(t, wb_ref[...]) + bb_ref[...]
    m = jnp.mean(z, axis=0, keepdims=True)
    v = jnp.mean((z - m) ** 2, axis=0, keepdims=True)
    return (z - m) / jnp.sqrt(v + 1e-5) * g_ref[...] + b_ref[...]


def _l1_body(x_ref, p_ref, wa_ref, ba_ref, wb_ref, bb_ref, g_ref, b_ref,
             wr_ref, br_ref, out_ref):
    x = x_ref[...]
    zn = _mlp_bn(p_ref, x, wa_ref, ba_ref, wb_ref, bb_ref, g_ref, b_ref)
    res = _mm(x, wr_ref[...]) + br_ref[...]
    out_ref[...] = jnp.maximum(res + zn, 0.0)


def _l2_body(x_ref, p_ref, wa_ref, ba_ref, wb_ref, bb_ref, g_ref, b_ref,
             out_ref):
    x = x_ref[...]
    zn = _mlp_bn(p_ref, x, wa_ref, ba_ref, wb_ref, bb_ref, g_ref, b_ref)
    out_ref[...] = jnp.maximum(x + zn, 0.0)


def _l3_body(x_ref, p_ref, wa_ref, ba_ref, wb_ref, bb_ref, g_ref, b_ref,
             wr_ref, br_ref, out_ref):
    x = x_ref[...]
    zn = _mlp_bn(p_ref, x, wa_ref, ba_ref, wb_ref, bb_ref, g_ref, b_ref)
    u = _mm(x, wr_ref[...]) + br_ref[...] + zn
    mx = jnp.max(u, axis=1, keepdims=True)
    lse = jnp.log(jnp.sum(jnp.exp(u - mx), axis=1, keepdims=True)) + mx
    out_ref[...] = u - lse


_OUT = jax.ShapeDtypeStruct((N, D), jnp.float32)
_l1_call = pl.pallas_call(_l1_body, out_shape=_OUT)
_l2_call = pl.pallas_call(_l2_body, out_shape=_OUT)
_l3_call = pl.pallas_call(_l3_body, out_shape=_OUT)


def kernel(x, edge_index, w1a, b1a, w1b, b1b, w2a, b2a, w2b, b2b,
           w3a, b3a, w3b, b3b, bn1_g, bn1_b, bn2_g, bn2_b, wr1, br1, wr2, br2):
    src = edge_index[0]
    dst = edge_index[1]
    E = src.shape[0]
    m = -(-(-(-E // CHUNK)) // (NS * SEG))
    m += 1 - (m % 2)
    cpt = m * SEG
    nrows = NS * cpt
    pad = nrows * CHUNK - E
    srcp = jnp.concatenate(
        [src, jnp.zeros((pad + SEG * CHUNK,), jnp.int32)]).reshape(-1, CHUNK)
    dstp = jnp.concatenate(
        [dst, jnp.full((pad + SEG * CHUNK,), N, jnp.int32)]).reshape(-1, CHUNK)

    r = lambda v: v.reshape(1, D)
    b1a_, b1b_, b2a_, b2b_, b3a_, b3b_ = map(r, (b1a, b1b, b2a, b2b, b3a, b3b))
    g1, bt1, g2, bt2 = map(r, (bn1_g, bn1_b, bn2_g, bn2_b))
    br1_, br2_ = r(br1), r(br2)

    p = _segment_partials(x, srcp, dstp, cpt)
    h1 = _l1_call(x, p, w1a, b1a_, w1b, b1b_, g1, bt1, wr1, br1_)
    p = _segment_partials(h1, srcp, dstp, cpt)
    h2 = _l2_call(h1, p, w2a, b2a_, w2b, b2b_, g1, bt1)
    p = _segment_partials(h2, srcp, dstp, cpt)
    return _l3_call(h2, p, w3a, b3a_, w3b, b3b_, g2, bt2, wr2, br2_)


# speedup vs baseline: 1.4048x; 1.4048x over previous
import functools

import jax
import jax.numpy as jnp
from jax import lax
from jax.experimental import pallas as pl
from jax.experimental.pallas import tpu as pltpu
from jax.experimental.pallas import tpu_sc as plsc

N = 10000
D = 128
NC = 2
NS = 16
CHUNK = 128
SEG_CPT = 16
C0_FRAC = 0.9
SLAB = 632
LAST = N - (NS - 1) * SLAB


@functools.lru_cache(maxsize=None)
def _make_seg_kernel(k0, k1):
    mesh = plsc.VectorSubcoreMesh(core_axis_name="c", subcore_axis_name="s")

    @functools.partial(
        pl.kernel,
        mesh=mesh,
        out_type=jax.ShapeDtypeStruct((NC * N, D), jnp.float32),
        scratch_types=[
            pltpu.VMEM_SHARED((N + 8, D), jnp.float32),
            pltpu.VMEM((SEG_CPT, CHUNK), jnp.int32),
            pltpu.VMEM((SEG_CPT, CHUNK), jnp.int32),
            pltpu.VMEM((CHUNK, D), jnp.float32),
            pltpu.VMEM((CHUNK, D), jnp.float32),
            pltpu.SemaphoreType.DMA,
            pltpu.SemaphoreType.DMA,
        ],
    )
    def seg(h_hbm, src_hbm, dst_hbm, out_hbm, acc, si, di, ra, rb, sa, sb):
        c = lax.axis_index("c")
        s = lax.axis_index("s")
        slab_off = pl.multiple_of(s * SLAB, 8)

        @pl.when(s < NS - 1)
        def _():
            pltpu.sync_copy(h_hbm.at[pl.ds(slab_off, SLAB)],
                            acc.at[pl.ds(slab_off, SLAB)])

        @pl.when(s == NS - 1)
        def _():
            pltpu.sync_copy(h_hbm.at[pl.ds((NS - 1) * SLAB, LAST)],
                            acc.at[pl.ds((NS - 1) * SLAB, LAST)])

        plsc.subcore_barrier()

        def run_segment(rowbase):
            pltpu.sync_copy(src_hbm.at[pl.ds(rowbase, SEG_CPT)], si)
            pltpu.sync_copy(dst_hbm.at[pl.ds(rowbase, SEG_CPT)], di)
            pltpu.async_copy(h_hbm.at[si.at[0]], ra, sa)

            @pl.loop(0, SEG_CPT, step=2)
            def _(j):
                pltpu.async_copy(h_hbm.at[si.at[j + 1]], rb, sb)
                pltpu.make_async_copy(h_hbm.at[si.at[j]], ra, sa).wait()
                pltpu.sync_copy(ra, acc.at[di.at[j]], add=True)

                @pl.when(j + 2 < SEG_CPT)
                def _():
                    pltpu.async_copy(h_hbm.at[si.at[j + 2]], ra, sa)

                pltpu.make_async_copy(h_hbm.at[si.at[j + 1]], rb, sb).wait()
                pltpu.sync_copy(rb, acc.at[di.at[j + 1]], add=True)

        @pl.when(c == 0)
        def _():
            @pl.loop(0, k0)
            def _(g):
                run_segment(pl.multiple_of((s * k0 + g) * SEG_CPT, 8))

        @pl.when(c == 1)
        def _():
            @pl.loop(0, k1)
            def _(g):
                run_segment(pl.multiple_of((NS * k0 + s * k1 + g) * SEG_CPT, 8))

        plsc.subcore_barrier()
        out_off = pl.multiple_of(c * N + s * SLAB, 8)

        @pl.when(s < NS - 1)
        def _():
            pltpu.sync_copy(acc.at[pl.ds(slab_off, SLAB)],
                            out_hbm.at[pl.ds(out_off, SLAB)])

        @pl.when(s == NS - 1)
        def _():
            pltpu.sync_copy(acc.at[pl.ds((NS - 1) * SLAB, LAST)],
                            out_hbm.at[pl.ds(pl.multiple_of(c * N + (NS - 1) * SLAB, 8), LAST)])

    return seg


def _segment_partials(h, srcp, dstp, k0, k1):
    return _make_seg_kernel(k0, k1)(h, srcp, dstp)


def _mm(a, w):
    return lax.dot_general(a, w, (((1,), (1,)), ((), ())),
                           preferred_element_type=jnp.float32,
                           precision=lax.Precision.HIGHEST)


def _mlp_bn(p_ref, x, wa_ref, ba_ref, wb_ref, bb_ref, g_ref, b_ref):
    y = p_ref[pl.ds(0, N), :] + p_ref[pl.ds(N, N), :] - x
    t = jnp.maximum(_mm(y, wa_ref[...]) + ba_ref[...], 0.0)
    z = _mm(t, wb_ref[...]) + bb_ref[...]
    m = jnp.mean(z, axis=0, keepdims=True)
    v = jnp.mean((z - m) ** 2, axis=0, keepdims=True)
    return (z - m) / jnp.sqrt(v + 1e-5) * g_ref[...] + b_ref[...]


def _l1_body(x_ref, p_ref, wa_ref, ba_ref, wb_ref, bb_ref, g_ref, b_ref,
             wr_ref, br_ref, out_ref):
    x = x_ref[...]
    zn = _mlp_bn(p_ref, x, wa_ref, ba_ref, wb_ref, bb_ref, g_ref, b_ref)
    res = _mm(x, wr_ref[...]) + br_ref[...]
    out_ref[...] = jnp.maximum(res + zn, 0.0)


def _l2_body(x_ref, p_ref, wa_ref, ba_ref, wb_ref, bb_ref, g_ref, b_ref,
             out_ref):
    x = x_ref[...]
    zn = _mlp_bn(p_ref, x, wa_ref, ba_ref, wb_ref, bb_ref, g_ref, b_ref)
    out_ref[...] = jnp.maximum(x + zn, 0.0)


def _l3_body(x_ref, p_ref, wa_ref, ba_ref, wb_ref, bb_ref, g_ref, b_ref,
             wr_ref, br_ref, out_ref):
    x = x_ref[...]
    zn = _mlp_bn(p_ref, x, wa_ref, ba_ref, wb_ref, bb_ref, g_ref, b_ref)
    u = _mm(x, wr_ref[...]) + br_ref[...] + zn
    mx = jnp.max(u, axis=1, keepdims=True)
    lse = jnp.log(jnp.sum(jnp.exp(u - mx), axis=1, keepdims=True)) + mx
    out_ref[...] = u - lse


_OUT = jax.ShapeDtypeStruct((N, D), jnp.float32)
_l1_call = pl.pallas_call(_l1_body, out_shape=_OUT)
_l2_call = pl.pallas_call(_l2_body, out_shape=_OUT)
_l3_call = pl.pallas_call(_l3_body, out_shape=_OUT)


def kernel(x, edge_index, w1a, b1a, w1b, b1b, w2a, b2a, w2b, b2b,
           w3a, b3a, w3b, b3b, bn1_g, bn1_b, bn2_g, bn2_b, wr1, br1, wr2, br2):
    src = edge_index[0]
    dst = edge_index[1]
    E = src.shape[0]
    tot = -(-E // CHUNK)
    u = -(-tot // (NS * SEG_CPT))
    k0 = max(1, min(u - 1, round(C0_FRAC * u + 1e-9)))
    k1 = u - k0
    nrows = u * NS * SEG_CPT
    pad = nrows * CHUNK - E
    if pad:
        srcp = jnp.concatenate([src, jnp.zeros((pad,), jnp.int32)])
        dstp = jnp.concatenate([dst, jnp.full((pad,), N, jnp.int32)])
    else:
        srcp, dstp = src, dst
    srcp = srcp.reshape(nrows, CHUNK)
    dstp = dstp.reshape(nrows, CHUNK)

    r = lambda v: v.reshape(1, D)
    b1a_, b1b_, b2a_, b2b_, b3a_, b3b_ = map(r, (b1a, b1b, b2a, b2b, b3a, b3b))
    g1, bt1, g2, bt2 = map(r, (bn1_g, bn1_b, bn2_g, bn2_b))
    br1_, br2_ = r(br1), r(br2)

    p = _segment_partials(x, srcp, dstp, k0, k1)
    h1 = _l1_call(x, p, w1a, b1a_, w1b, b1b_, g1, bt1, wr1, br1_)
    p = _segment_partials(h1, srcp, dstp, k0, k1)
    h2 = _l2_call(h1, p, w2a, b2a_, w2b, b2b_, g1, bt1)
    p = _segment_partials(h2, srcp, dstp, k0, k1)
    return _l3_call(h2, p, w3a, b3a_, w3b, b3b_, g2, bt2, wr2, br2_)
